# baseline (speedup 1.0000x reference)
import functools

import jax
import jax.numpy as jnp
from jax import lax
from jax.experimental import pallas as pl
from jax.experimental.pallas import tpu as pltpu
from jax.experimental.pallas import tpu_sc as plsc

B = 16384
L = 200
V = 1000000
D = 32
H = 64
NLAB = 10

NW = 32
SPW = B // NW
G = 32
NCH = SPW // G
VL = 16


def _sc_pool(shp_ids, col_ids, shape_table, color_table):
    mesh = plsc.VectorSubcoreMesh(core_axis_name="c", subcore_axis_name="s")

    @functools.partial(
        pl.kernel,
        out_type=jax.ShapeDtypeStruct((B, 2 * D), jnp.float32),
        mesh=mesh,
        compiler_params=pltpu.CompilerParams(use_tc_tiling_on_sc=False),
        scratch_types=[
            pltpu.VMEM((G * L,), jnp.int32),
            pltpu.VMEM((G * L,), jnp.int32),
            pltpu.VMEM((L, D), jnp.float32),
            pltpu.VMEM((L, D), jnp.float32),
            pltpu.VMEM((L, D), jnp.float32),
            pltpu.VMEM((L, D), jnp.float32),
            pltpu.VMEM((G, 2 * D), jnp.float32),
            pltpu.SemaphoreType.DMA,
            pltpu.SemaphoreType.DMA,
            pltpu.SemaphoreType.DMA,
            pltpu.SemaphoreType.DMA,
        ],
    )
    def pool(shp_hbm, col_hbm, stab_hbm, ctab_hbm, out_hbm,
             idx_s, idx_c, rows_s0, rows_s1, rows_c0, rows_c1, feat_v,
             sem_s0, sem_s1, sem_c0, sem_c1):
        wid = lax.axis_index("s") * 2 + lax.axis_index("c")
        wbase = wid * SPW
        bufs = ((rows_s0, rows_c0, sem_s0, sem_c0),
                (rows_s1, rows_c1, sem_s1, sem_c1))

        def descs(g, b):
            rs, rc, ss, sc = bufs[b]
            return (
                pltpu.make_async_copy(
                    stab_hbm.at[idx_s.at[pl.ds(g * L, 128)]],
                    rs.at[pl.ds(0, 128)], ss),
                pltpu.make_async_copy(
                    stab_hbm.at[idx_s.at[pl.ds(g * L + 128, L - 128)]],
                    rs.at[pl.ds(128, L - 128)], ss),
                pltpu.make_async_copy(
                    ctab_hbm.at[idx_c.at[pl.ds(g * L, 128)]],
                    rc.at[pl.ds(0, 128)], sc),
                pltpu.make_async_copy(
                    ctab_hbm.at[idx_c.at[pl.ds(g * L + 128, L - 128)]],
                    rc.at[pl.ds(128, L - 128)], sc),
            )

        def issue(g, b):
            for d in descs(g, b):
                d.start()

        def wait(g, b):
            for d in descs(g, b):
                d.wait()

        def accumulate(g, b):
            rs, rc, _, _ = bufs[b]

            def acc_body(l, carry):
                a0, a1, c0, c1 = carry
                a0 = a0 + rs[l, pl.ds(0, VL)]
                a1 = a1 + rs[l, pl.ds(VL, VL)]
                c0 = c0 + rc[l, pl.ds(0, VL)]
                c1 = c1 + rc[l, pl.ds(VL, VL)]
                return (a0, a1, c0, c1)

            z = jnp.zeros((VL,), jnp.float32)
            a0, a1, c0, c1 = lax.fori_loop(0, L, acc_body, (z, z, z, z),
                                           unroll=4)
            feat_v[g, pl.ds(0, VL)] = a0
            feat_v[g, pl.ds(VL, VL)] = a1
            feat_v[g, pl.ds(2 * VL, VL)] = c0
            feat_v[g, pl.ds(3 * VL, VL)] = c1

        def chunk_body(ch, _):
            ebase = pl.multiple_of((wbase + ch * G) * L, 8)
            pltpu.sync_copy(shp_hbm.at[pl.ds(ebase, G * L)], idx_s)
            pltpu.sync_copy(col_hbm.at[pl.ds(ebase, G * L)], idx_c)

            def mask_body(i, _):
                sl = pl.ds(pl.multiple_of(i * VL, 8), VL)
                s = idx_s[sl]
                c = idx_c[sl]
                idx_c[sl] = jnp.where(s == 0, 0, c)
                return 0

            lax.fori_loop(0, G * L // VL, mask_body, 0, unroll=8)

            issue(0, 0)
            issue(1, 1)

            def pair_body(pp, _):
                for b in range(2):
                    g = pp * 2 + b
                    wait(g, b)
                    accumulate(g, b)
                    issue(g + 2, b)
                return 0

            lax.fori_loop(0, G // 2 - 1, pair_body, 0)
            for b in range(2):
                g = G - 2 + b
                wait(g, b)
                accumulate(g, b)

            pltpu.sync_copy(feat_v, out_hbm.at[pl.ds(wbase + ch * G, G)])
            return 0

        lax.fori_loop(0, NCH, chunk_body, 0)

    return pool(shp_ids, col_ids, shape_table, color_table)


def _mlp(feat_raw, lens_col, W1, b1, W2, b2):
    Bb = 2048

    def body(feat_ref, lens_ref, w1_ref, b1_ref, w2_ref, b2_ref, out_ref):
        f = feat_ref[...]
        inv = 1.0 / lens_ref[...]
        h = jnp.dot(f, w1_ref[...], preferred_element_type=jnp.float32)
        h = jnp.maximum(h * inv + b1_ref[...], 0.0)
        out_ref[...] = (
            jnp.dot(h, w2_ref[...], preferred_element_type=jnp.float32)
            + b2_ref[...]
        )

    return pl.pallas_call(
        body,
        grid=(B // Bb,),
        in_specs=[
            pl.BlockSpec((Bb, 2 * D), lambda i: (i, 0)),
            pl.BlockSpec((Bb, 1), lambda i: (i, 0)),
            pl.BlockSpec((2 * D, H), lambda i: (0, 0)),
            pl.BlockSpec((1, H), lambda i: (0, 0)),
            pl.BlockSpec((H, NLAB), lambda i: (0, 0)),
            pl.BlockSpec((1, NLAB), lambda i: (0, 0)),
        ],
        out_specs=pl.BlockSpec((Bb, NLAB), lambda i: (i, 0)),
        out_shape=jax.ShapeDtypeStruct((B, NLAB), jnp.float32),
    )(feat_raw, lens_col, W1, b1.reshape(1, H), W2, b2.reshape(1, NLAB))


def kernel(shp_ids, col_ids, lens, shape_table, color_table, W1, b1, W2, b2):
    feat = _sc_pool(shp_ids.reshape(-1), col_ids.reshape(-1),
                    shape_table, color_table)
    lens_col = lens.astype(jnp.float32).reshape(B, 1)
    return _mlp(feat, lens_col, W1, b1, W2, b2)


# speedup vs baseline: 1.0150x; 1.0150x over previous
import functools

import jax
import jax.numpy as jnp
from jax import lax
from jax.experimental import pallas as pl
from jax.experimental.pallas import tpu as pltpu
from jax.experimental.pallas import tpu_sc as plsc

B = 16384
L = 200
V = 1000000
D = 32
H = 64
NLAB = 10

NW = 32
G = 32
VL = 16
NSPLIT = 4


def _sc_pool(shp_ids, col_ids, shape_table, color_table, bs):
    spw = bs // NW
    nch = spw // G
    mesh = plsc.VectorSubcoreMesh(core_axis_name="c", subcore_axis_name="s")

    @functools.partial(
        pl.kernel,
        out_type=jax.ShapeDtypeStruct((bs, 2 * D), jnp.float32),
        mesh=mesh,
        compiler_params=pltpu.CompilerParams(use_tc_tiling_on_sc=False),
        scratch_types=[
            pltpu.VMEM((G, L), jnp.int32),
            pltpu.VMEM((G, L), jnp.int32),
            pltpu.VMEM((L, D), jnp.float32),
            pltpu.VMEM((L, D), jnp.float32),
            pltpu.VMEM((L, D), jnp.float32),
            pltpu.VMEM((L, D), jnp.float32),
            pltpu.VMEM((G, 2 * D), jnp.float32),
            pltpu.SemaphoreType.DMA,
            pltpu.SemaphoreType.DMA,
            pltpu.SemaphoreType.DMA,
            pltpu.SemaphoreType.DMA,
        ],
    )
    def pool(shp_hbm, col_hbm, stab_hbm, ctab_hbm, out_hbm,
             idx_s, idx_c, rows_s0, rows_s1, rows_c0, rows_c1, feat_v,
             sem_s0, sem_s1, sem_c0, sem_c1):
        wid = lax.axis_index("s") * 2 + lax.axis_index("c")
        wbase = wid * spw
        bufs = ((rows_s0, rows_c0, sem_s0, sem_c0),
                (rows_s1, rows_c1, sem_s1, sem_c1))

        def descs(g, b):
            rs, rc, ss, sc = bufs[b]
            return (
                pltpu.make_async_copy(
                    stab_hbm.at[idx_s.at[g, pl.ds(0, 128)]],
                    rs.at[pl.ds(0, 128)], ss),
                pltpu.make_async_copy(
                    stab_hbm.at[idx_s.at[g, pl.ds(128, L - 128)]],
                    rs.at[pl.ds(128, L - 128)], ss),
                pltpu.make_async_copy(
                    ctab_hbm.at[idx_c.at[g, pl.ds(0, 128)]],
                    rc.at[pl.ds(0, 128)], sc),
                pltpu.make_async_copy(
                    ctab_hbm.at[idx_c.at[g, pl.ds(128, L - 128)]],
                    rc.at[pl.ds(128, L - 128)], sc),
            )

        def issue(g, b):
            for d in descs(g, b):
                d.start()

        def wait(g, b):
            for d in descs(g, b):
                d.wait()

        def accumulate(g, b):
            rs, rc, _, _ = bufs[b]

            def acc_body(l, carry):
                a0, a1, c0, c1 = carry
                a0 = a0 + rs[l, pl.ds(0, VL)]
                a1 = a1 + rs[l, pl.ds(VL, VL)]
                c0 = c0 + rc[l, pl.ds(0, VL)]
                c1 = c1 + rc[l, pl.ds(VL, VL)]
                return (a0, a1, c0, c1)

            z = jnp.zeros((VL,), jnp.float32)
            a0, a1, c0, c1 = lax.fori_loop(0, L, acc_body, (z, z, z, z),
                                           unroll=4)
            feat_v[g, pl.ds(0, VL)] = a0
            feat_v[g, pl.ds(VL, VL)] = a1
            feat_v[g, pl.ds(2 * VL, VL)] = c0
            feat_v[g, pl.ds(3 * VL, VL)] = c1

        def chunk_body(ch, _):
            rbase = pl.multiple_of(wbase + ch * G, 8)
            pltpu.sync_copy(shp_hbm.at[pl.ds(rbase, G)], idx_s)
            pltpu.sync_copy(col_hbm.at[pl.ds(rbase, G)], idx_c)

            offs = tuple(range(0, L - VL, VL)) + (L - VL,)

            def mask_body(g, _):
                for o in offs:
                    sl = pl.ds(pl.multiple_of(o, 8), VL)
                    s = idx_s[g, sl]
                    c = idx_c[g, sl]
                    idx_c[g, sl] = jnp.where(s == 0, 0, c)
                return 0

            lax.fori_loop(0, G, mask_body, 0)

            issue(0, 0)
            issue(1, 1)

            def pair_body(pp, _):
                for b in range(2):
                    g = pp * 2 + b
                    wait(g, b)
                    accumulate(g, b)
                    issue(g + 2, b)
                return 0

            lax.fori_loop(0, G // 2 - 1, pair_body, 0)
            for b in range(2):
                g = G - 2 + b
                wait(g, b)
                accumulate(g, b)

            pltpu.sync_copy(feat_v, out_hbm.at[pl.ds(wbase + ch * G, G)])
            return 0

        lax.fori_loop(0, nch, chunk_body, 0)

    return pool(shp_ids, col_ids, shape_table, color_table)


def _mlp(feat_raw, lens_col, W1, b1, W2, b2):
    Bb = 2048

    def body(feat_ref, lens_ref, w1_ref, b1_ref, w2_ref, b2_ref, out_ref):
        f = feat_ref[...]
        inv = 1.0 / lens_ref[...]
        h = jnp.dot(f, w1_ref[...], preferred_element_type=jnp.float32)
        h = jnp.maximum(h * inv + b1_ref[...], 0.0)
        out_ref[...] = (
            jnp.dot(h, w2_ref[...], preferred_element_type=jnp.float32)
            + b2_ref[...]
        )

    return pl.pallas_call(
        body,
        grid=(B // Bb,),
        in_specs=[
            pl.BlockSpec((Bb, 2 * D), lambda i: (i, 0)),
            pl.BlockSpec((Bb, 1), lambda i: (i, 0)),
            pl.BlockSpec((2 * D, H), lambda i: (0, 0)),
            pl.BlockSpec((1, H), lambda i: (0, 0)),
            pl.BlockSpec((H, NLAB), lambda i: (0, 0)),
            pl.BlockSpec((1, NLAB), lambda i: (0, 0)),
        ],
        out_specs=pl.BlockSpec((Bb, NLAB), lambda i: (i, 0)),
        out_shape=jax.ShapeDtypeStruct((B, NLAB), jnp.float32),
    )(feat_raw, lens_col, W1, b1.reshape(1, H), W2, b2.reshape(1, NLAB))


def kernel(shp_ids, col_ids, lens, shape_table, color_table, W1, b1, W2, b2):
    bs = B // NSPLIT
    feats = [
        _sc_pool(shp_ids[i * bs:(i + 1) * bs], col_ids[i * bs:(i + 1) * bs],
                 shape_table, color_table, bs)
        for i in range(NSPLIT)
    ]
    feat = jnp.concatenate(feats, axis=0)
    lens_col = lens.astype(jnp.float32).reshape(B, 1)
    return _mlp(feat, lens_col, W1, b1, W2, b2)


# speedup vs baseline: 1.0153x; 1.0003x over previous
import functools

import jax
import jax.numpy as jnp
from jax import lax
from jax.experimental import pallas as pl
from jax.experimental.pallas import tpu as pltpu
from jax.experimental.pallas import tpu_sc as plsc

B = 16384
L = 200
V = 1000000
D = 32
H = 64
NLAB = 10

LH = 128
LT = L - LH
NW = 32
G = 32
VL = 16


def _split_ids(shp_ids, col_ids):
    BR = 512

    def body(s_ref, c_ref, sh_ref, st_ref, ch_ref, ct_ref):
        s = s_ref[...]
        c = c_ref[...]
        cm = jnp.where(s == 0, 0, c)
        sh_ref[...] = s[:, :LH]
        ch_ref[...] = cm[:, :LH]
        zpad = jnp.zeros((BR, LH - LT), jnp.int32)
        st_ref[...] = jnp.concatenate([s[:, LH:], zpad], axis=1)
        ct_ref[...] = jnp.concatenate([cm[:, LH:], zpad], axis=1)

    return pl.pallas_call(
        body,
        grid=(B // BR,),
        in_specs=[
            pl.BlockSpec((BR, L), lambda i: (i, 0)),
            pl.BlockSpec((BR, L), lambda i: (i, 0)),
        ],
        out_specs=[
            pl.BlockSpec((BR, LH), lambda i: (i, 0)),
            pl.BlockSpec((BR, LH), lambda i: (i, 0)),
            pl.BlockSpec((BR, LH), lambda i: (i, 0)),
            pl.BlockSpec((BR, LH), lambda i: (i, 0)),
        ],
        out_shape=[
            jax.ShapeDtypeStruct((B, LH), jnp.int32),
            jax.ShapeDtypeStruct((B, LH), jnp.int32),
            jax.ShapeDtypeStruct((B, LH), jnp.int32),
            jax.ShapeDtypeStruct((B, LH), jnp.int32),
        ],
    )(shp_ids, col_ids)


def _sc_pool(sh, st, ch, ct, shape_table, color_table):
    spw = B // NW
    nch = spw // G
    mesh = plsc.VectorSubcoreMesh(core_axis_name="c", subcore_axis_name="s")

    @functools.partial(
        pl.kernel,
        out_type=jax.ShapeDtypeStruct((B, 2 * D), jnp.float32),
        mesh=mesh,
        compiler_params=pltpu.CompilerParams(use_tc_tiling_on_sc=False),
        scratch_types=[
            pltpu.VMEM((G, LH), jnp.int32),
            pltpu.VMEM((G, LH), jnp.int32),
            pltpu.VMEM((G, LH), jnp.int32),
            pltpu.VMEM((G, LH), jnp.int32),
            pltpu.VMEM((L, D), jnp.float32),
            pltpu.VMEM((L, D), jnp.float32),
            pltpu.VMEM((L, D), jnp.float32),
            pltpu.VMEM((L, D), jnp.float32),
            pltpu.VMEM((G, 2 * D), jnp.float32),
            pltpu.SemaphoreType.DMA,
            pltpu.SemaphoreType.DMA,
            pltpu.SemaphoreType.DMA,
            pltpu.SemaphoreType.DMA,
        ],
    )
    def pool(sh_hbm, st_hbm, ch_hbm, ct_hbm, stab_hbm, ctab_hbm, out_hbm,
             idx_sh, idx_st, idx_ch, idx_ct,
             rows_s0, rows_s1, rows_c0, rows_c1, feat_v,
             sem_s0, sem_s1, sem_c0, sem_c1):
        wid = lax.axis_index("s") * 2 + lax.axis_index("c")
        wbase = wid * spw
        bufs = ((rows_s0, rows_c0, sem_s0, sem_c0),
                (rows_s1, rows_c1, sem_s1, sem_c1))

        def descs(g, b):
            rs, rc, ss, sc = bufs[b]
            return (
                pltpu.make_async_copy(
                    stab_hbm.at[idx_sh.at[g]],
                    rs.at[pl.ds(0, LH)], ss),
                pltpu.make_async_copy(
                    stab_hbm.at[idx_st.at[g, pl.ds(0, LT)]],
                    rs.at[pl.ds(LH, LT)], ss),
                pltpu.make_async_copy(
                    ctab_hbm.at[idx_ch.at[g]],
                    rc.at[pl.ds(0, LH)], sc),
                pltpu.make_async_copy(
                    ctab_hbm.at[idx_ct.at[g, pl.ds(0, LT)]],
                    rc.at[pl.ds(LH, LT)], sc),
            )

        def issue(g, b):
            for d in descs(g, b):
                d.start()

        def wait(g, b):
            for d in descs(g, b):
                d.wait()

        def accumulate(g, b):
            rs, rc, _, _ = bufs[b]

            def acc_body(l, carry):
                a0, a1, c0, c1 = carry
                a0 = a0 + rs[l, pl.ds(0, VL)]
                a1 = a1 + rs[l, pl.ds(VL, VL)]
                c0 = c0 + rc[l, pl.ds(0, VL)]
                c1 = c1 + rc[l, pl.ds(VL, VL)]
                return (a0, a1, c0, c1)

            z = jnp.zeros((VL,), jnp.float32)
            a0, a1, c0, c1 = lax.fori_loop(0, L, acc_body, (z, z, z, z),
                                           unroll=4)
            feat_v[g, pl.ds(0, VL)] = a0
            feat_v[g, pl.ds(VL, VL)] = a1
            feat_v[g, pl.ds(2 * VL, VL)] = c0
            feat_v[g, pl.ds(3 * VL, VL)] = c1

        def chunk_body(ch_i, _):
            rbase = pl.multiple_of(wbase + ch_i * G, 8)
            pltpu.sync_copy(sh_hbm.at[pl.ds(rbase, G)], idx_sh)
            pltpu.sync_copy(st_hbm.at[pl.ds(rbase, G)], idx_st)
            pltpu.sync_copy(ch_hbm.at[pl.ds(rbase, G)], idx_ch)
            pltpu.sync_copy(ct_hbm.at[pl.ds(rbase, G)], idx_ct)


            issue(0, 0)
            issue(1, 1)

            def pair_body(pp, _):
                for b in range(2):
                    g = pp * 2 + b
                    wait(g, b)
                    accumulate(g, b)
                    issue(g + 2, b)
                return 0

            lax.fori_loop(0, G // 2 - 1, pair_body, 0)
            for b in range(2):
                g = G - 2 + b
                wait(g, b)
                accumulate(g, b)

            pltpu.sync_copy(feat_v, out_hbm.at[pl.ds(wbase + ch_i * G, G)])
            return 0

        lax.fori_loop(0, nch, chunk_body, 0)

    return pool(sh, st, ch, ct, shape_table, color_table)


def _mlp(feat_raw, lens_col, W1, b1, W2, b2):
    Bb = 2048

    def body(feat_ref, lens_ref, w1_ref, b1_ref, w2_ref, b2_ref, out_ref):
        f = feat_ref[...]
        inv = 1.0 / lens_ref[...]
        h = jnp.dot(f, w1_ref[...], preferred_element_type=jnp.float32)
        h = jnp.maximum(h * inv + b1_ref[...], 0.0)
        out_ref[...] = (
            jnp.dot(h, w2_ref[...], preferred_element_type=jnp.float32)
            + b2_ref[...]
        )

    return pl.pallas_call(
        body,
        grid=(B // Bb,),
        in_specs=[
            pl.BlockSpec((Bb, 2 * D), lambda i: (i, 0)),
            pl.BlockSpec((Bb, 1), lambda i: (i, 0)),
            pl.BlockSpec((2 * D, H), lambda i: (0, 0)),
            pl.BlockSpec((1, H), lambda i: (0, 0)),
            pl.BlockSpec((H, NLAB), lambda i: (0, 0)),
            pl.BlockSpec((1, NLAB), lambda i: (0, 0)),
        ],
        out_specs=pl.BlockSpec((Bb, NLAB), lambda i: (i, 0)),
        out_shape=jax.ShapeDtypeStruct((B, NLAB), jnp.float32),
    )(feat_raw, lens_col, W1, b1.reshape(1, H), W2, b2.reshape(1, NLAB))


def kernel(shp_ids, col_ids, lens, shape_table, color_table, W1, b1, W2, b2):
    sh, st, ch, ct = _split_ids(shp_ids, col_ids)
    feat = _sc_pool(sh, st, ch, ct, shape_table, color_table)
    lens_col = lens.astype(jnp.float32).reshape(B, 1)
    return _mlp(feat, lens_col, W1, b1, W2, b2)


# speedup vs baseline: 1.0708x; 1.0547x over previous
import functools

import jax
import jax.numpy as jnp
from jax import lax
from jax.experimental import pallas as pl
from jax.experimental.pallas import tpu as pltpu
from jax.experimental.pallas import tpu_sc as plsc

B = 16384
L = 200
V = 1000000
D = 32
H = 64
NLAB = 10

LH = 128
LT = L - LH
NW = 32
G = 32
VL = 16


def _split_ids(shp_ids, col_ids):
    BR = 512

    def body(s_ref, c_ref, sh_ref, st_ref, ch_ref, ct_ref):
        s = s_ref[...]
        c = c_ref[...]
        cm = jnp.where(s == 0, 0, c)
        sh_ref[...] = s[:, :LH]
        ch_ref[...] = cm[:, :LH]
        zpad = jnp.zeros((BR, LH - LT), jnp.int32)
        st_ref[...] = jnp.concatenate([s[:, LH:], zpad], axis=1)
        ct_ref[...] = jnp.concatenate([cm[:, LH:], zpad], axis=1)

    return pl.pallas_call(
        body,
        grid=(B // BR,),
        in_specs=[
            pl.BlockSpec((BR, L), lambda i: (i, 0)),
            pl.BlockSpec((BR, L), lambda i: (i, 0)),
        ],
        out_specs=[
            pl.BlockSpec((BR, LH), lambda i: (i, 0)),
            pl.BlockSpec((BR, LH), lambda i: (i, 0)),
            pl.BlockSpec((BR, LH), lambda i: (i, 0)),
            pl.BlockSpec((BR, LH), lambda i: (i, 0)),
        ],
        out_shape=[
            jax.ShapeDtypeStruct((B, LH), jnp.int32),
            jax.ShapeDtypeStruct((B, LH), jnp.int32),
            jax.ShapeDtypeStruct((B, LH), jnp.int32),
            jax.ShapeDtypeStruct((B, LH), jnp.int32),
        ],
    )(shp_ids, col_ids)


def _sc_pool(idh, idt, table, tag):
    spw = B // NW
    nch = spw // G
    mesh = plsc.VectorSubcoreMesh(core_axis_name="c", subcore_axis_name="s")

    @functools.partial(
        pl.kernel,
        out_type=jax.ShapeDtypeStruct((B, D), jnp.float32),
        mesh=mesh,
        compiler_params=pltpu.CompilerParams(use_tc_tiling_on_sc=False),
        scratch_types=[
            pltpu.VMEM((G, LH), jnp.int32),
            pltpu.VMEM((G, LH), jnp.int32),
            pltpu.VMEM((L, D), jnp.float32),
            pltpu.VMEM((L, D), jnp.float32),
            pltpu.VMEM((G, D), jnp.float32),
            pltpu.SemaphoreType.DMA,
            pltpu.SemaphoreType.DMA,
        ],
    )
    def pool(idh_hbm, idt_hbm, tab_hbm, out_hbm,
             idx_h, idx_t, rows_0, rows_1, feat_v, sem_0, sem_1):
        wid = lax.axis_index("s") * 2 + lax.axis_index("c")
        wbase = wid * spw
        bufs = ((rows_0, sem_0), (rows_1, sem_1))

        def descs(g, b):
            rows, sem = bufs[b]
            return (
                pltpu.make_async_copy(
                    tab_hbm.at[idx_h.at[g]],
                    rows.at[pl.ds(0, LH)], sem),
                pltpu.make_async_copy(
                    tab_hbm.at[idx_t.at[g, pl.ds(0, LT)]],
                    rows.at[pl.ds(LH, LT)], sem),
            )

        def issue(g, b):
            for d in descs(g, b):
                d.start()

        def wait(g, b):
            for d in descs(g, b):
                d.wait()

        def accumulate(g, b):
            rows, _ = bufs[b]

            def acc_body(l, carry):
                a0, a1 = carry
                a0 = a0 + rows[l, pl.ds(0, VL)]
                a1 = a1 + rows[l, pl.ds(VL, VL)]
                return (a0, a1)

            z = jnp.zeros((VL,), jnp.float32)
            a0, a1 = lax.fori_loop(0, L, acc_body, (z, z), unroll=8)
            feat_v[g, pl.ds(0, VL)] = a0
            feat_v[g, pl.ds(VL, VL)] = a1

        def chunk_body(ch_i, _):
            rbase = pl.multiple_of(wbase + ch_i * G, 8)
            pltpu.sync_copy(idh_hbm.at[pl.ds(rbase, G)], idx_h)
            pltpu.sync_copy(idt_hbm.at[pl.ds(rbase, G)], idx_t)


            issue(0, 0)
            issue(1, 1)

            def pair_body(pp, _):
                for b in range(2):
                    g = pp * 2 + b
                    wait(g, b)
                    accumulate(g, b)
                    issue(g + 2, b)
                return 0

            lax.fori_loop(0, G // 2 - 1, pair_body, 0)
            for b in range(2):
                g = G - 2 + b
                wait(g, b)
                accumulate(g, b)

            pltpu.sync_copy(feat_v, out_hbm.at[pl.ds(wbase + ch_i * G, G)])
            return 0

        lax.fori_loop(0, nch, chunk_body, 0)

    del tag
    return pool(idh, idt, table)


def _mlp(feat_s, feat_c, lens_col, W1, b1, W2, b2):
    Bb = 2048

    def body(fs_ref, fc_ref, lens_ref, w1s_ref, w1c_ref, b1_ref, w2_ref,
             b2_ref, out_ref):
        inv = 1.0 / lens_ref[...]
        h = (jnp.dot(fs_ref[...], w1s_ref[...],
                     preferred_element_type=jnp.float32)
             + jnp.dot(fc_ref[...], w1c_ref[...],
                       preferred_element_type=jnp.float32))
        h = jnp.maximum(h * inv + b1_ref[...], 0.0)
        out_ref[...] = (
            jnp.dot(h, w2_ref[...], preferred_element_type=jnp.float32)
            + b2_ref[...]
        )

    return pl.pallas_call(
        body,
        grid=(B // Bb,),
        in_specs=[
            pl.BlockSpec((Bb, D), lambda i: (i, 0)),
            pl.BlockSpec((Bb, D), lambda i: (i, 0)),
            pl.BlockSpec((Bb, 1), lambda i: (i, 0)),
            pl.BlockSpec((D, H), lambda i: (0, 0)),
            pl.BlockSpec((D, H), lambda i: (0, 0)),
            pl.BlockSpec((1, H), lambda i: (0, 0)),
            pl.BlockSpec((H, NLAB), lambda i: (0, 0)),
            pl.BlockSpec((1, NLAB), lambda i: (0, 0)),
        ],
        out_specs=pl.BlockSpec((Bb, NLAB), lambda i: (i, 0)),
        out_shape=jax.ShapeDtypeStruct((B, NLAB), jnp.float32),
    )(feat_s, feat_c, lens_col, W1[:D], W1[D:], b1.reshape(1, H), W2,
      b2.reshape(1, NLAB))


def kernel(shp_ids, col_ids, lens, shape_table, color_table, W1, b1, W2, b2):
    sh, st, ch, ct = _split_ids(shp_ids, col_ids)
    feat_s = _sc_pool(sh, st, shape_table, "shape")
    feat_c = _sc_pool(ch, ct, color_table, "color")
    lens_col = lens.astype(jnp.float32).reshape(B, 1)
    return _mlp(feat_s, feat_c, lens_col, W1, b1, W2, b2)
